# Initial kernel scaffold; baseline (speedup 1.0000x reference)
import jax
import jax.numpy as jnp
from jax.experimental import pallas as pl


def kernel(occ, hold, connection, W_occ, W_hold, W_con):
    raise NotImplementedError("write your pallas kernel here")


# speedup vs baseline: 20.5151x; 20.5151x over previous
import jax
import jax.numpy as jnp
from jax import lax
from jax.experimental import pallas as pl
from jax.experimental.pallas import tpu as pltpu
from jax.experimental.pallas import tpu_sc as plsc

B, L = 16384, 200
N = B * L
NW = 32
PER_W = N // NW
C = 2048
KS = C // 128
STEPS = C // 16
NCH = PER_W // C
ROWS128 = N // 128


def _body(occ_h, hold_h, con_h, wocc_h, wh_h, wc_h, out_h,
          idxo_v, idxh_v, idxc_v, outb_v, tblh_v, tblc_v, sem):
    cid = lax.axis_index("c")
    sid = lax.axis_index("s")
    wid = sid * 2 + cid

    pltpu.sync_copy(wh_h, tblh_v)
    pltpu.sync_copy(wc_h, tblc_v)

    iota = lax.iota(jnp.int32, 16)
    col4 = jnp.full((16,), 4, jnp.int32)
    col5 = jnp.full((16,), 5, jnp.int32)
    col6 = jnp.full((16,), 6, jnp.int32)
    col7 = jnp.full((16,), 7, jnp.int32)

    def chunk(ch, carry):
        base = pl.multiple_of(wid * PER_W + ch * C, C)
        row_base = pl.multiple_of(base // 128, KS)

        pltpu.sync_copy(occ_h.at[pl.ds(row_base, KS)], idxo_v)
        pltpu.sync_copy(hold_h.at[pl.ds(base, C)], idxh_v)
        pltpu.sync_copy(con_h.at[pl.ds(base, C)], idxc_v)

        cps = [
            pltpu.async_copy(wocc_h.at[idxo_v.at[j]],
                             outb_v.at[pl.ds(j * 128, 128)], sem)
            for j in range(KS)
        ]
        for cp in cps:
            cp.wait()

        def step(s, c2):
            r0 = s * 16
            rowv = iota + r0
            ih2 = idxh_v[pl.ds(r0, 16)] * 2
            h0 = plsc.load_gather(tblh_v, [ih2])
            h1 = plsc.load_gather(tblh_v, [ih2 + 1])
            plsc.store_scatter(outb_v, [rowv, col4], h0)
            plsc.store_scatter(outb_v, [rowv, col5], h1)
            ic2 = idxc_v[pl.ds(r0, 16)] * 2
            c0 = plsc.load_gather(tblc_v, [ic2])
            c1 = plsc.load_gather(tblc_v, [ic2 + 1])
            plsc.store_scatter(outb_v, [rowv, col6], c0)
            plsc.store_scatter(outb_v, [rowv, col7], c1)
            return c2

        lax.fori_loop(0, STEPS, step, 0, unroll=False)

        pltpu.sync_copy(outb_v, out_h.at[pl.ds(base, C)])
        return carry

    lax.fori_loop(0, NCH, chunk, 0, unroll=False)


@jax.jit
def _sc_call(occ2, holdf, conf, wocc8, whf, wcf):
    mesh = plsc.VectorSubcoreMesh(core_axis_name="c", subcore_axis_name="s")
    return pl.kernel(
        _body,
        out_type=jax.ShapeDtypeStruct((N, 8), jnp.float32),
        mesh=mesh,
        scratch_types=[
            pltpu.VMEM((KS, 128), jnp.int32),
            pltpu.VMEM((C,), jnp.int32),
            pltpu.VMEM((C,), jnp.int32),
            pltpu.VMEM((C, 8), jnp.float32),
            pltpu.VMEM((2000,), jnp.float32),
            pltpu.VMEM((2000,), jnp.float32),
            pltpu.SemaphoreType.DMA,
        ],
        compiler_params=pltpu.CompilerParams(
            use_tc_tiling_on_sc=False, needs_layout_passes=False),
    )(occ2, holdf, conf, wocc8, whf, wcf)


def kernel(occ, hold, connection, W_occ, W_hold, W_con):
    occ2 = occ.reshape(ROWS128, 128).astype(jnp.int32)
    holdf = hold.reshape(N).astype(jnp.int32)
    conf = connection.reshape(N).astype(jnp.int32)
    wocc8 = jnp.pad(W_occ, ((0, 0), (0, 4)))
    whf = W_hold.reshape(2000)
    wcf = W_con.reshape(2000)
    out = _sc_call(occ2, holdf, conf, wocc8, whf, wcf)
    return out.reshape(B, L, 8)


# speedup vs baseline: 20.7440x; 1.0112x over previous
import jax
import jax.numpy as jnp
from jax import lax
from jax.experimental import pallas as pl
from jax.experimental.pallas import tpu as pltpu
from jax.experimental.pallas import tpu_sc as plsc

B, L = 16384, 200
N = B * L
NW = 32
PER_W = N // NW
C = 3200
BPC = C // L
KS = C // 128
STEPS = C // 16
NCH = PER_W // C
ROWS128 = N // 128


def _body(occ_h, hold_h, con_h, wocc_h, wh_h, wc_h, out_h,
          idxo_v, idxh_v, idxc_v, outb_v, tblh_v, tblc_v, sem, osem):
    cid = lax.axis_index("c")
    sid = lax.axis_index("s")
    wid = sid * 2 + cid

    pltpu.sync_copy(wh_h, tblh_v)
    pltpu.sync_copy(wc_h, tblc_v)

    iota = lax.iota(jnp.int32, 16)
    col4 = jnp.full((16,), 4, jnp.int32)
    col5 = jnp.full((16,), 5, jnp.int32)
    col6 = jnp.full((16,), 6, jnp.int32)
    col7 = jnp.full((16,), 7, jnp.int32)

    def chunk(ch, carry):
        base = pl.multiple_of(wid * PER_W + ch * C, C)
        row_base = pl.multiple_of(base // 128, KS)
        b0 = pl.multiple_of(base // L, BPC)

        pltpu.sync_copy(occ_h.at[pl.ds(row_base, KS)], idxo_v)
        pltpu.sync_copy(hold_h.at[pl.ds(base, C)], idxh_v)
        pltpu.sync_copy(con_h.at[pl.ds(base, C)], idxc_v)

        cps = [
            pltpu.async_copy(wocc_h.at[idxo_v.at[j]],
                             outb_v.at[pl.ds(j * 128, 128)], sem)
            for j in range(KS)
        ]
        for cp in cps:
            cp.wait()

        def step(s, c2):
            r0 = s * 16
            rowv = iota + r0
            ih2 = idxh_v[pl.ds(r0, 16)] * 2
            h0 = plsc.load_gather(tblh_v, [ih2])
            h1 = plsc.load_gather(tblh_v, [ih2 + 1])
            plsc.store_scatter(outb_v, [rowv, col4], h0)
            plsc.store_scatter(outb_v, [rowv, col5], h1)
            ic2 = idxc_v[pl.ds(r0, 16)] * 2
            c0 = plsc.load_gather(tblc_v, [ic2])
            c1 = plsc.load_gather(tblc_v, [ic2 + 1])
            plsc.store_scatter(outb_v, [rowv, col6], c0)
            plsc.store_scatter(outb_v, [rowv, col7], c1)
            return c2

        lax.fori_loop(0, STEPS, step, 0, unroll=False)

        ocps = [
            pltpu.async_copy(outb_v.at[pl.ds(i * L, L)],
                             out_h.at[b0 + i], osem)
            for i in range(BPC)
        ]
        for cp in ocps:
            cp.wait()
        return carry

    lax.fori_loop(0, NCH, chunk, 0, unroll=False)


@jax.jit
def _sc_call(occ2, holdf, conf, wocc8, whf, wcf):
    mesh = plsc.VectorSubcoreMesh(core_axis_name="c", subcore_axis_name="s")
    return pl.kernel(
        _body,
        out_type=jax.ShapeDtypeStruct((B, L, 8), jnp.float32),
        mesh=mesh,
        scratch_types=[
            pltpu.VMEM((KS, 128), jnp.int32),
            pltpu.VMEM((C,), jnp.int32),
            pltpu.VMEM((C,), jnp.int32),
            pltpu.VMEM((C, 8), jnp.float32),
            pltpu.VMEM((2000,), jnp.float32),
            pltpu.VMEM((2000,), jnp.float32),
            pltpu.SemaphoreType.DMA,
            pltpu.SemaphoreType.DMA,
        ],
        compiler_params=pltpu.CompilerParams(
            use_tc_tiling_on_sc=False, needs_layout_passes=False),
    )(occ2, holdf, conf, wocc8, whf, wcf)


def kernel(occ, hold, connection, W_occ, W_hold, W_con):
    occ2 = occ.reshape(ROWS128, 128).astype(jnp.int32)
    holdf = hold.reshape(N).astype(jnp.int32)
    conf = connection.reshape(N).astype(jnp.int32)
    wocc8 = jnp.pad(W_occ, ((0, 0), (0, 4)))
    whf = W_hold.reshape(2000)
    wcf = W_con.reshape(2000)
    return _sc_call(occ2, holdf, conf, wocc8, whf, wcf)
